# baseline (speedup 1.0000x reference)
import functools

import jax
import jax.numpy as jnp
from jax import lax
from jax.experimental import pallas as pl
from jax.experimental.pallas import tpu as pltpu
from jax.experimental.pallas import tpu_sc as plsc

VOCAB = 1000000
DIM = 64
BATCH = 4096
HIST = 200

NC, NS = 2, 16
NW = NC * NS
BW = BATCH // NW
DT = DIM // 8

VBLK = 128
NFULL = VOCAB // VBLK
VTAIL = VOCAB - NFULL * VBLK


def _tr_body(src_hbm, tail_hbm, dst_hbm, in0, in1, ot0, ot1, sem_i0, sem_i1,
             sem_o0, sem_o1):
    wid = lax.axis_index("s") * NC + lax.axis_index("c")
    nblk = NFULL // NW + jnp.where(wid < NFULL % NW, 1, 0)

    ins = (in0, in1)
    ots = (ot0, ot1)
    sem_i = (sem_i0, sem_i1)
    sem_o = (sem_o0, sem_o1)

    lane = lax.iota(jnp.int32, 16)
    dvecs = [(lane + 16 * g) % DIM for g in range(2 * DIM // 16)]
    steps = [(lane + 16 * g) // DIM for g in range(2 * DIM // 16)]

    def v0_of(i):
        return pl.multiple_of((wid + i * NW) * VBLK, VBLK)

    def start_in(i, b):
        pltpu.async_copy(src_hbm.at[:, pl.ds(v0_of(i), VBLK)], ins[b],
                         sem_i[b])

    def start_out(i, b):
        pltpu.async_copy(ots[b], dst_hbm.at[pl.ds(pl.multiple_of(v0_of(i) // 2, DIM), DIM)],
                         sem_o[b])

    def wait_in(b):
        pltpu.make_async_copy(src_hbm.at[:, pl.ds(0, VBLK)], ins[b],
                              sem_i[b]).wait()

    def wait_out(b):
        pltpu.make_async_copy(ots[b], dst_hbm.at[pl.ds(0, DIM)],
                              sem_o[b]).wait()

    def transpose(b, nrow):
        def row(k, _):
            vals = [plsc.load_gather(ins[b], [dvecs[g], steps[g] + 2 * k])
                    for g in range(2 * DIM // 16)]
            for g in range(2 * DIM // 16):
                ots[b][k, pl.ds(16 * g, 16)] = vals[g]
            return _

        lax.fori_loop(0, nrow, row, 0)

    @pl.when(nblk > 0)
    def _run():
        start_in(0, 0)

        @pl.when(nblk > 1)
        def _p2():
            start_in(1, 1)

        def half(i, b):
            wait_in(b)

            @pl.when(i >= 2)
            def _w():
                wait_out(b)

            transpose(b, DIM)

            @pl.when(i + 2 < nblk)
            def _n():
                start_in(i + 2, b)

            start_out(i, b)

        def stepper(i, _):
            @pl.when(2 * i < nblk)
            def _a():
                half(2 * i, 0)

            @pl.when(2 * i + 1 < nblk)
            def _b():
                half(2 * i + 1, 1)

            return _

        lax.fori_loop(0, (nblk + 1) // 2, stepper, 0)
        wait_out(0)

        @pl.when(nblk > 1)
        def _d2():
            wait_out(1)

    @pl.when(wid == 0)
    def _tail():
        pltpu.sync_copy(tail_hbm, in0)

        def trow(k, _):
            vals = [plsc.load_gather(in0, [steps[g] + 2 * k, dvecs[g]])
                    for g in range(2 * DIM // 16)]
            for g in range(2 * DIM // 16):
                ot0[k, pl.ds(16 * g, 16)] = vals[g]
            return _

        lax.fori_loop(0, VTAIL // 2, trow, 0)
        pltpu.sync_copy(ot0.at[pl.ds(0, VTAIL // 2)],
                        dst_hbm.at[pl.ds(NFULL * VBLK // 2, VTAIL // 2)])


@jax.jit
def _transpose_table(src, tail):
    mesh = plsc.VectorSubcoreMesh(core_axis_name="c", subcore_axis_name="s")
    return pl.kernel(
        _tr_body,
        out_type=jax.ShapeDtypeStruct((VOCAB // 2, 2 * DIM), jnp.float32),
        mesh=mesh,
        compiler_params=pltpu.CompilerParams(needs_layout_passes=False),
        scratch_types=[
            pltpu.VMEM((DIM, VBLK), jnp.float32),
            pltpu.VMEM((DIM, VBLK), jnp.float32),
            pltpu.VMEM((DIM, 2 * DIM), jnp.float32),
            pltpu.VMEM((DIM, 2 * DIM), jnp.float32),
            pltpu.SemaphoreType.DMA,
            pltpu.SemaphoreType.DMA,
            pltpu.SemaphoreType.DMA,
            pltpu.SemaphoreType.DMA,
        ],
    )(src, tail)


def _embed_body(idx_hbm, table_hbm, out_hbm, idx_v, rows0, rows1, rows2,
                rows3, tb0, tb1, sem_g0, sem_g1, sem_g2, sem_g3, sem_o0,
                sem_o1):
    wid = lax.axis_index("s") * NC + lax.axis_index("c")
    b0 = wid * BW
    pltpu.sync_copy(idx_hbm.at[:, pl.ds(b0, BW)], idx_v)

    rows = (rows0, rows1, rows2, rows3)
    tbs = (tb0, tb1)
    sem_g = (sem_g0, sem_g1, sem_g2, sem_g3)
    sem_o = (sem_o0, sem_o1)

    lane = lax.iota(jnp.int32, 16)
    bvecs = [lane + 16 * g for g in range(BW // 16)]

    def start_gather(h, b):
        pltpu.async_copy(table_hbm.at[idx_v.at[h]], rows[b], sem_g[b])

    def start_out(h, b):
        pltpu.async_copy(tbs[b], out_hbm.at[h, :, wid], sem_o[b])

    def wait_gather(b):
        pltpu.make_async_copy(table_hbm.at[pl.ds(0, BW)], rows[b],
                              sem_g[b]).wait()

    def wait_out(b):
        pltpu.make_async_copy(tbs[b], out_hbm.at[0, :, wid], sem_o[b]).wait()

    def transpose(b, t):
        def dtloop(dt, _):
            for ds in range(8):
                dvec = jnp.full((16,), 0, jnp.int32) + (dt * 8 + ds)
                vals = [plsc.load_gather(rows[b], [bvecs[g], dvec])
                        for g in range(BW // 16)]
                for g in range(BW // 16):
                    tbs[t][dt, ds, pl.ds(16 * g, 16)] = vals[g]
            return _

        lax.fori_loop(0, DT, dtloop, 0)

    start_gather(0, 0)
    start_gather(1, 1)
    start_gather(2, 2)

    def quarter(h, b, t):
        wait_gather(b)

        @pl.when(h >= 2)
        def _w():
            wait_out(t)

        transpose(b, t)

        @pl.when(h + 3 < HIST)
        def _g():
            start_gather(h + 3, (b + 3) % 4)

        start_out(h, t)

    def step(i, _):
        h = 4 * i
        quarter(h, 0, 0)
        quarter(h + 1, 1, 1)
        quarter(h + 2, 2, 0)
        quarter(h + 3, 3, 1)
        return _

    lax.fori_loop(0, HIST // 4, step, 0)
    wait_out(0)
    wait_out(1)


@jax.jit
def _embed(idx_t, table):
    mesh = plsc.VectorSubcoreMesh(core_axis_name="c", subcore_axis_name="s")
    return pl.kernel(
        _embed_body,
        out_type=jax.ShapeDtypeStruct((HIST, DT, NW, 8, BW), jnp.float32),
        mesh=mesh,
        compiler_params=pltpu.CompilerParams(use_tc_tiling_on_sc=False,
                                            needs_layout_passes=False),
        scratch_types=[
            pltpu.VMEM((HIST, BW), jnp.int32),
            pltpu.VMEM((BW, DIM), jnp.float32),
            pltpu.VMEM((BW, DIM), jnp.float32),
            pltpu.VMEM((BW, DIM), jnp.float32),
            pltpu.VMEM((BW, DIM), jnp.float32),
            pltpu.VMEM((DT, 8, BW), jnp.float32),
            pltpu.VMEM((DT, 8, BW), jnp.float32),
            pltpu.SemaphoreType.DMA,
            pltpu.SemaphoreType.DMA,
            pltpu.SemaphoreType.DMA,
            pltpu.SemaphoreType.DMA,
            pltpu.SemaphoreType.DMA,
            pltpu.SemaphoreType.DMA,
        ],
    )(idx_t, table)


def kernel(inp, embeddings):
    idx_t = inp.T.astype(jnp.int32)
    emb_t = embeddings.T
    tail = jnp.pad(embeddings[VOCAB - VTAIL:, :], ((0, 0), (0, DIM)))
    t2 = _transpose_table(emb_t, tail)
    table = t2.reshape(VOCAB, DIM)
    out5 = _embed(idx_t, table)
    return out5.transpose(2, 4, 0, 1, 3).reshape(BATCH, HIST, DIM)


# speedup vs baseline: 1.8886x; 1.8886x over previous
import functools

import jax
import jax.numpy as jnp
from jax import lax
from jax.experimental import pallas as pl
from jax.experimental.pallas import tpu as pltpu
from jax.experimental.pallas import tpu_sc as plsc

VOCAB = 1000000
DIM = 64
BATCH = 4096
HIST = 200

NC, NS = 2, 16
NW = NC * NS
BW = BATCH // NW
DT = DIM // 8
PITCH = DIM + 1


def _embed_body(idx_hbm, table_hbm, out_hbm, idx_v, rows0, rows1, rows2,
                rows3, rp, tb0, tb1, sem_g0, sem_g1, sem_g2, sem_g3, sem_o0,
                sem_o1):
    wid = lax.axis_index("s") * NC + lax.axis_index("c")
    b0 = wid * BW
    pltpu.sync_copy(idx_hbm.at[:, pl.ds(b0, BW)], idx_v)

    rows = (rows0, rows1, rows2, rows3)
    tbs = (tb0, tb1)
    sem_g = (sem_g0, sem_g1, sem_g2, sem_g3)
    sem_o = (sem_o0, sem_o1)

    lane = lax.iota(jnp.int32, 16)
    bvecs = [lane * PITCH + 16 * PITCH * g for g in range(BW // 16)]

    def start_gather(h, b):
        pltpu.async_copy(table_hbm.at[idx_v.at[h]], rows[b], sem_g[b])

    def start_out(h, t):
        pltpu.async_copy(tbs[t], out_hbm.at[h, :, wid], sem_o[t])

    def wait_gather(b):
        pltpu.make_async_copy(table_hbm.at[pl.ds(0, BW)], rows[b],
                              sem_g[b]).wait()

    def wait_out(t):
        pltpu.make_async_copy(tbs[t], out_hbm.at[0, :, wid], sem_o[t]).wait()

    def transpose(b, t):
        def pack(r, _):
            o = r * PITCH
            for k in range(DIM // 16):
                rp[pl.ds(o + 16 * k, 16)] = rows[b][r, pl.ds(16 * k, 16)]
            return _

        lax.fori_loop(0, BW, pack, 0)

        def dtloop(dt, _):
            for ds in range(8):
                dvec = jnp.full((16,), 0, jnp.int32) + (dt * 8 + ds)
                vals = [plsc.load_gather(rp, [bvecs[g] + dvec])
                        for g in range(BW // 16)]
                for g in range(BW // 16):
                    tbs[t][dt, ds, pl.ds(16 * g, 16)] = vals[g]
            return _

        lax.fori_loop(0, DT, dtloop, 0)

    start_gather(0, 0)
    start_gather(1, 1)
    start_gather(2, 2)

    def quarter(h, b, t):
        wait_gather(b)

        @pl.when(h >= 2)
        def _w():
            wait_out(t)

        transpose(b, t)

        @pl.when(h + 3 < HIST)
        def _g():
            start_gather(h + 3, (b + 3) % 4)

        start_out(h, t)

    def step(i, _):
        h = 4 * i
        quarter(h, 0, 0)
        quarter(h + 1, 1, 1)
        quarter(h + 2, 2, 0)
        quarter(h + 3, 3, 1)
        return _

    lax.fori_loop(0, HIST // 4, step, 0)
    wait_out(0)
    wait_out(1)


@jax.jit
def _embed(idx_t, table):
    mesh = plsc.VectorSubcoreMesh(core_axis_name="c", subcore_axis_name="s")
    return pl.kernel(
        _embed_body,
        out_type=jax.ShapeDtypeStruct((HIST, DT, NW, 8, BW), jnp.float32),
        mesh=mesh,
        compiler_params=pltpu.CompilerParams(use_tc_tiling_on_sc=False,
                                            needs_layout_passes=False),
        scratch_types=[
            pltpu.VMEM((HIST, BW), jnp.int32),
            pltpu.VMEM((BW, DIM), jnp.float32),
            pltpu.VMEM((BW, DIM), jnp.float32),
            pltpu.VMEM((BW, DIM), jnp.float32),
            pltpu.VMEM((BW, DIM), jnp.float32),
            pltpu.VMEM((BW * PITCH,), jnp.float32),
            pltpu.VMEM((DT, 8, BW), jnp.float32),
            pltpu.VMEM((DT, 8, BW), jnp.float32),
            pltpu.SemaphoreType.DMA,
            pltpu.SemaphoreType.DMA,
            pltpu.SemaphoreType.DMA,
            pltpu.SemaphoreType.DMA,
            pltpu.SemaphoreType.DMA,
            pltpu.SemaphoreType.DMA,
        ],
    )(idx_t, table)


def kernel(inp, embeddings):
    idx_t = inp.T.astype(jnp.int32)
    out5 = _embed(idx_t, embeddings)
    return out5.transpose(2, 4, 0, 1, 3).reshape(BATCH, HIST, DIM)


# speedup vs baseline: 1.8991x; 1.0056x over previous
import functools

import jax
import jax.numpy as jnp
from jax import lax
from jax.experimental import pallas as pl
from jax.experimental.pallas import tpu as pltpu
from jax.experimental.pallas import tpu_sc as plsc

VOCAB = 1000000
DIM = 64
BATCH = 4096
HIST = 200

NC, NS = 2, 16
NW = NC * NS
BW = BATCH // NW
DT = DIM // 8
PITCH = DIM + 1
PITCH2 = 129


VBLK = 128
NFULL = VOCAB // VBLK
VTAIL = VOCAB - NFULL * VBLK


def _tr_body(src_hbm, tail_hbm, dst_hbm, in0, in1, rp2, ot0, ot1, sem_i0,
             sem_i1, sem_o0, sem_o1):
    wid = lax.axis_index("s") * NC + lax.axis_index("c")
    nblk = NFULL // NW + jnp.where(wid < NFULL % NW, 1, 0)

    ins = (in0, in1)
    ots = (ot0, ot1)
    sem_i = (sem_i0, sem_i1)
    sem_o = (sem_o0, sem_o1)

    lane = lax.iota(jnp.int32, 16)
    base_g = [((lane + 16 * g) % DIM) * PITCH2 + (lane + 16 * g) // DIM
              for g in range(2 * DIM // 16)]

    def v0_of(i):
        return pl.multiple_of((wid + i * NW) * VBLK, VBLK)

    def start_in(i, b):
        pltpu.async_copy(src_hbm.at[:, pl.ds(v0_of(i), VBLK)], ins[b],
                         sem_i[b])

    def start_out(i, b):
        pltpu.async_copy(
            ots[b],
            dst_hbm.at[pl.ds(pl.multiple_of(v0_of(i) // 2, DIM), DIM)],
            sem_o[b])

    def wait_in(b):
        pltpu.make_async_copy(src_hbm.at[:, pl.ds(0, VBLK)], ins[b],
                              sem_i[b]).wait()

    def wait_out(b):
        pltpu.make_async_copy(ots[b], dst_hbm.at[pl.ds(0, DIM)],
                              sem_o[b]).wait()

    def transpose(b):
        def pack(r, _):
            o = r * PITCH2
            for k in range(VBLK // 16):
                rp2[pl.ds(o + 16 * k, 16)] = ins[b][r, pl.ds(16 * k, 16)]
            return _

        lax.fori_loop(0, DIM, pack, 0)

        def row(k, _):
            vals = [plsc.load_gather(rp2, [base_g[g] + 2 * k])
                    for g in range(2 * DIM // 16)]
            for g in range(2 * DIM // 16):
                ots[b][k, pl.ds(16 * g, 16)] = vals[g]
            return _

        lax.fori_loop(0, DIM, row, 0)

    start_in(0, 0)
    start_in(1, 1)

    def half(i, b):
        wait_in(b)

        @pl.when(i >= 2)
        def _w():
            wait_out(b)

        transpose(b)

        @pl.when(i + 2 < nblk)
        def _n():
            start_in(i + 2, b)

        start_out(i, b)

    def stepper(i, _):
        half(2 * i, 0)

        @pl.when(2 * i + 1 < nblk)
        def _b():
            half(2 * i + 1, 1)

        return _

    lax.fori_loop(0, (nblk + 1) // 2, stepper, 0)
    wait_out(0)
    wait_out(1)

    @pl.when(wid == 0)
    def _tail():
        pltpu.sync_copy(tail_hbm, in0)

        def pack(r, _):
            o = r * PITCH2
            for k in range(VBLK // 16):
                rp2[pl.ds(o + 16 * k, 16)] = in0[r, pl.ds(16 * k, 16)]
            return _

        lax.fori_loop(0, DIM, pack, 0)
        tbase = [((lane + 16 * g) // DIM) * PITCH2 + (lane + 16 * g) % DIM
                 for g in range(2 * DIM // 16)]

        def trow(k, _):
            vals = [plsc.load_gather(rp2, [tbase[g] + 2 * k * PITCH2])
                    for g in range(2 * DIM // 16)]
            for g in range(2 * DIM // 16):
                ot0[k, pl.ds(16 * g, 16)] = vals[g]
            return _

        lax.fori_loop(0, VTAIL // 2, trow, 0)
        pltpu.sync_copy(ot0.at[pl.ds(0, VTAIL // 2)],
                        dst_hbm.at[pl.ds(NFULL * VBLK // 2, VTAIL // 2)])


@jax.jit
def _transpose_table(src, tail):
    mesh = plsc.VectorSubcoreMesh(core_axis_name="c", subcore_axis_name="s")
    return pl.kernel(
        _tr_body,
        out_type=jax.ShapeDtypeStruct((VOCAB // 2, 2 * DIM), jnp.float32),
        mesh=mesh,
        compiler_params=pltpu.CompilerParams(needs_layout_passes=False),
        scratch_types=[
            pltpu.VMEM((DIM, VBLK), jnp.float32),
            pltpu.VMEM((DIM, VBLK), jnp.float32),
            pltpu.VMEM((DIM * PITCH2,), jnp.float32),
            pltpu.VMEM((DIM, 2 * DIM), jnp.float32),
            pltpu.VMEM((DIM, 2 * DIM), jnp.float32),
            pltpu.SemaphoreType.DMA,
            pltpu.SemaphoreType.DMA,
            pltpu.SemaphoreType.DMA,
            pltpu.SemaphoreType.DMA,
        ],
    )(src, tail)


def _embed_body(idx_hbm, table_hbm, out_hbm, idx_v, rows0, rows1, rows2,
                rows3, rp, tb0, tb1, sem_g0, sem_g1, sem_g2, sem_g3, sem_o0,
                sem_o1):
    wid = lax.axis_index("s") * NC + lax.axis_index("c")
    b0 = wid * BW
    pltpu.sync_copy(idx_hbm.at[:, pl.ds(b0, BW)], idx_v)

    rows = (rows0, rows1, rows2, rows3)
    tbs = (tb0, tb1)
    sem_g = (sem_g0, sem_g1, sem_g2, sem_g3)
    sem_o = (sem_o0, sem_o1)

    lane = lax.iota(jnp.int32, 16)
    bvecs = [lane * PITCH + 16 * PITCH * g for g in range(BW // 16)]

    def start_gather(h, b):
        pltpu.async_copy(table_hbm.at[idx_v.at[h]], rows[b], sem_g[b])

    def start_out(h, t):
        pltpu.async_copy(tbs[t], out_hbm.at[h, :, wid], sem_o[t])

    def wait_gather(b):
        pltpu.make_async_copy(table_hbm.at[pl.ds(0, BW)], rows[b],
                              sem_g[b]).wait()

    def wait_out(t):
        pltpu.make_async_copy(tbs[t], out_hbm.at[0, :, wid], sem_o[t]).wait()

    def transpose(b, t):
        def pack(r, _):
            o = r * PITCH
            for k in range(DIM // 16):
                rp[pl.ds(o + 16 * k, 16)] = rows[b][r, pl.ds(16 * k, 16)]
            return _

        lax.fori_loop(0, BW, pack, 0)

        def dtloop(dt, _):
            for ds in range(8):
                dvec = jnp.full((16,), 0, jnp.int32) + (dt * 8 + ds)
                vals = [plsc.load_gather(rp, [bvecs[g] + dvec])
                        for g in range(BW // 16)]
                for g in range(BW // 16):
                    tbs[t][dt, ds, pl.ds(16 * g, 16)] = vals[g]
            return _

        lax.fori_loop(0, DT, dtloop, 0)

    start_gather(0, 0)
    start_gather(1, 1)
    start_gather(2, 2)

    def quarter(h, b, t):
        wait_gather(b)

        @pl.when(h >= 2)
        def _w():
            wait_out(t)

        transpose(b, t)

        @pl.when(h + 3 < HIST)
        def _g():
            start_gather(h + 3, (b + 3) % 4)

        start_out(h, t)

    def step(i, _):
        h = 4 * i
        quarter(h, 0, 0)
        quarter(h + 1, 1, 1)
        quarter(h + 2, 2, 0)
        quarter(h + 3, 3, 1)
        return _

    lax.fori_loop(0, HIST // 4, step, 0)
    wait_out(0)
    wait_out(1)


@jax.jit
def _embed(idx_t, table):
    mesh = plsc.VectorSubcoreMesh(core_axis_name="c", subcore_axis_name="s")
    return pl.kernel(
        _embed_body,
        out_type=jax.ShapeDtypeStruct((HIST, DT, NW, 8, BW), jnp.float32),
        mesh=mesh,
        compiler_params=pltpu.CompilerParams(use_tc_tiling_on_sc=False,
                                            needs_layout_passes=False),
        scratch_types=[
            pltpu.VMEM((HIST, BW), jnp.int32),
            pltpu.VMEM((BW, DIM), jnp.float32),
            pltpu.VMEM((BW, DIM), jnp.float32),
            pltpu.VMEM((BW, DIM), jnp.float32),
            pltpu.VMEM((BW, DIM), jnp.float32),
            pltpu.VMEM((BW * PITCH,), jnp.float32),
            pltpu.VMEM((DT, 8, BW), jnp.float32),
            pltpu.VMEM((DT, 8, BW), jnp.float32),
            pltpu.SemaphoreType.DMA,
            pltpu.SemaphoreType.DMA,
            pltpu.SemaphoreType.DMA,
            pltpu.SemaphoreType.DMA,
            pltpu.SemaphoreType.DMA,
            pltpu.SemaphoreType.DMA,
        ],
    )(idx_t, table)


def kernel(inp, embeddings):
    idx_t = inp.T.astype(jnp.int32)
    emb_t = embeddings.T
    tail = jnp.pad(embeddings[VOCAB - VTAIL:, :], ((0, 0), (0, DIM)))
    t2 = _transpose_table(emb_t, tail)
    table = t2.reshape(VOCAB, DIM)
    out5 = _embed(idx_t, table)
    return out5.transpose(2, 4, 0, 1, 3).reshape(BATCH, HIST, DIM)


# speedup vs baseline: 3.6577x; 1.9260x over previous
import functools

import jax
import jax.numpy as jnp
from jax import lax
from jax.experimental import pallas as pl
from jax.experimental.pallas import tpu as pltpu
from jax.experimental.pallas import tpu_sc as plsc

VOCAB = 1000000
DIM = 64
BATCH = 4096
HIST = 200

NC, NS = 2, 16
NW = NC * NS
BW = BATCH // NW
DT = DIM // 8
PITCH = DIM + 1
PITCH2 = 129


VBLK = 128
NFULL = VOCAB // VBLK
VTAIL = VOCAB - NFULL * VBLK


def _tr_body(src_hbm, tail_hbm, dst_hbm, in0, in1, rp2, ot0, ot1, sem_i0,
             sem_i1, sem_o0, sem_o1):
    wid = lax.axis_index("s") * NC + lax.axis_index("c")
    nblk = NFULL // NW + jnp.where(wid < NFULL % NW, 1, 0)

    ins = (in0, in1)
    ots = (ot0, ot1)
    sem_i = (sem_i0, sem_i1)
    sem_o = (sem_o0, sem_o1)

    lane = lax.iota(jnp.int32, 16)
    base_g = [((lane + 16 * g) % DIM) * PITCH2 + (lane + 16 * g) // DIM
              for g in range(2 * DIM // 16)]

    def v0_of(i):
        return pl.multiple_of((wid + i * NW) * VBLK, VBLK)

    def start_in(i, b):
        pltpu.async_copy(src_hbm.at[:, pl.ds(v0_of(i), VBLK)], ins[b],
                         sem_i[b])

    def start_out(i, b):
        pltpu.async_copy(
            ots[b],
            dst_hbm.at[pl.ds(pl.multiple_of(v0_of(i) // 2, DIM), DIM)],
            sem_o[b])

    def wait_in(b):
        pltpu.make_async_copy(src_hbm.at[:, pl.ds(0, VBLK)], ins[b],
                              sem_i[b]).wait()

    def wait_out(b):
        pltpu.make_async_copy(ots[b], dst_hbm.at[pl.ds(0, DIM)],
                              sem_o[b]).wait()

    def transpose(b):
        def pack(r, _):
            o = r * PITCH2
            vals = [ins[b][r, pl.ds(16 * k, 16)] for k in range(VBLK // 16)]
            for k in range(VBLK // 16):
                rp2[pl.ds(o + 16 * k, 16)] = vals[k]
            return _

        lax.fori_loop(0, DIM, pack, 0)

        def row(k, _):
            vals = [plsc.load_gather(rp2, [base_g[g] + 2 * k])
                    for g in range(2 * DIM // 16)]
            for g in range(2 * DIM // 16):
                ots[b][k, pl.ds(16 * g, 16)] = vals[g]
            return _

        lax.fori_loop(0, DIM, row, 0)

    start_in(0, 0)
    start_in(1, 1)

    def half(i, b):
        wait_in(b)

        @pl.when(i >= 2)
        def _w():
            wait_out(b)

        transpose(b)

        @pl.when(i + 2 < nblk)
        def _n():
            start_in(i + 2, b)

        start_out(i, b)

    def stepper(i, _):
        half(2 * i, 0)

        @pl.when(2 * i + 1 < nblk)
        def _b():
            half(2 * i + 1, 1)

        return _

    lax.fori_loop(0, (nblk + 1) // 2, stepper, 0)
    wait_out(0)
    wait_out(1)

    @pl.when(wid == 0)
    def _tail():
        pltpu.sync_copy(tail_hbm, in0)

        def pack(r, _):
            o = r * PITCH2
            vals = [in0[r, pl.ds(16 * k, 16)] for k in range(VBLK // 16)]
            for k in range(VBLK // 16):
                rp2[pl.ds(o + 16 * k, 16)] = vals[k]
            return _

        lax.fori_loop(0, DIM, pack, 0)
        tbase = [((lane + 16 * g) // DIM) * PITCH2 + (lane + 16 * g) % DIM
                 for g in range(2 * DIM // 16)]

        def trow(k, _):
            vals = [plsc.load_gather(rp2, [tbase[g] + 2 * k * PITCH2])
                    for g in range(2 * DIM // 16)]
            for g in range(2 * DIM // 16):
                ot0[k, pl.ds(16 * g, 16)] = vals[g]
            return _

        lax.fori_loop(0, VTAIL // 2, trow, 0)
        pltpu.sync_copy(ot0.at[pl.ds(0, VTAIL // 2)],
                        dst_hbm.at[pl.ds(NFULL * VBLK // 2, VTAIL // 2)])


@jax.jit
def _transpose_table(src, tail):
    mesh = plsc.VectorSubcoreMesh(core_axis_name="c", subcore_axis_name="s")
    return pl.kernel(
        _tr_body,
        out_type=jax.ShapeDtypeStruct((VOCAB // 2, 2 * DIM), jnp.float32),
        mesh=mesh,
        compiler_params=pltpu.CompilerParams(needs_layout_passes=False),
        scratch_types=[
            pltpu.VMEM((DIM, VBLK), jnp.float32),
            pltpu.VMEM((DIM, VBLK), jnp.float32),
            pltpu.VMEM((DIM * PITCH2,), jnp.float32),
            pltpu.VMEM((DIM, 2 * DIM), jnp.float32),
            pltpu.VMEM((DIM, 2 * DIM), jnp.float32),
            pltpu.SemaphoreType.DMA,
            pltpu.SemaphoreType.DMA,
            pltpu.SemaphoreType.DMA,
            pltpu.SemaphoreType.DMA,
        ],
    )(src, tail)


def _embed_body(idx_hbm, table_hbm, out_hbm, idx_v, rows0, rows1, rows2,
                rows3, rp, tb0, tb1, sem_g0, sem_g1, sem_g2, sem_g3, sem_o0,
                sem_o1):
    wid = lax.axis_index("s") * NC + lax.axis_index("c")
    b0 = wid * BW
    pltpu.sync_copy(idx_hbm.at[:, pl.ds(b0, BW)], idx_v)

    rows = (rows0, rows1, rows2, rows3)
    tbs = (tb0, tb1)
    sem_g = (sem_g0, sem_g1, sem_g2, sem_g3)
    sem_o = (sem_o0, sem_o1)

    lane = lax.iota(jnp.int32, 16)
    bvecs = [lane * PITCH + 16 * PITCH * g for g in range(BW // 16)]

    def start_gather(h, b):
        pltpu.async_copy(table_hbm.at[idx_v.at[h]], rows[b], sem_g[b])

    def start_out(h, t):
        pltpu.async_copy(tbs[t], out_hbm.at[h, :, wid], sem_o[t])

    def wait_gather(b):
        pltpu.make_async_copy(table_hbm.at[pl.ds(0, BW)], rows[b],
                              sem_g[b]).wait()

    def wait_out(t):
        pltpu.make_async_copy(tbs[t], out_hbm.at[0, :, wid], sem_o[t]).wait()

    def transpose(b, t):
        def pack(i, _):
            for j in range(2):
                r = 2 * i + j
                o = r * PITCH
                vals = [rows[b][r, pl.ds(16 * k, 16)]
                        for k in range(DIM // 16)]
                for k in range(DIM // 16):
                    rp[pl.ds(o + 16 * k, 16)] = vals[k]
            return _

        lax.fori_loop(0, BW // 2, pack, 0)

        def dtloop(dt, _):
            for ds in range(8):
                dvec = jnp.full((16,), 0, jnp.int32) + (dt * 8 + ds)
                vals = [plsc.load_gather(rp, [bvecs[g] + dvec])
                        for g in range(BW // 16)]
                for g in range(BW // 16):
                    tbs[t][dt, ds, pl.ds(16 * g, 16)] = vals[g]
            return _

        lax.fori_loop(0, DT, dtloop, 0)

    start_gather(0, 0)
    start_gather(1, 1)
    start_gather(2, 2)

    def quarter(h, b, t):
        wait_gather(b)

        @pl.when(h >= 2)
        def _w():
            wait_out(t)

        transpose(b, t)

        @pl.when(h + 3 < HIST)
        def _g():
            start_gather(h + 3, (b + 3) % 4)

        start_out(h, t)

    def step(i, _):
        h = 4 * i
        quarter(h, 0, 0)
        quarter(h + 1, 1, 1)
        quarter(h + 2, 2, 0)
        quarter(h + 3, 3, 1)
        return _

    lax.fori_loop(0, HIST // 4, step, 0)
    wait_out(0)
    wait_out(1)


@jax.jit
def _embed(idx_t, table):
    mesh = plsc.VectorSubcoreMesh(core_axis_name="c", subcore_axis_name="s")
    return pl.kernel(
        _embed_body,
        out_type=jax.ShapeDtypeStruct((HIST, DT, NW, 8, BW), jnp.float32),
        mesh=mesh,
        compiler_params=pltpu.CompilerParams(use_tc_tiling_on_sc=False,
                                            needs_layout_passes=False),
        scratch_types=[
            pltpu.VMEM((HIST, BW), jnp.int32),
            pltpu.VMEM((BW, DIM), jnp.float32),
            pltpu.VMEM((BW, DIM), jnp.float32),
            pltpu.VMEM((BW, DIM), jnp.float32),
            pltpu.VMEM((BW, DIM), jnp.float32),
            pltpu.VMEM((BW * PITCH,), jnp.float32),
            pltpu.VMEM((DT, 8, BW), jnp.float32),
            pltpu.VMEM((DT, 8, BW), jnp.float32),
            pltpu.SemaphoreType.DMA,
            pltpu.SemaphoreType.DMA,
            pltpu.SemaphoreType.DMA,
            pltpu.SemaphoreType.DMA,
            pltpu.SemaphoreType.DMA,
            pltpu.SemaphoreType.DMA,
        ],
    )(idx_t, table)


def kernel(inp, embeddings):
    idx_t = inp.T.astype(jnp.int32)
    emb_t = embeddings.T
    tail = jnp.pad(embeddings[VOCAB - VTAIL:, :], ((0, 0), (0, DIM)))
    t2 = _transpose_table(emb_t, tail)
    table = t2.reshape(VOCAB, DIM)
    out5 = _embed(idx_t, table)
    return out5.transpose(2, 4, 0, 1, 3).reshape(BATCH, HIST, DIM)
